# baseline (speedup 1.0000x reference)
import functools

import jax
import jax.numpy as jnp
import numpy as np
from jax import lax
from jax.experimental import pallas as pl
from jax.experimental.pallas import tpu as pltpu
from jax.experimental.pallas import tpu_sc as plsc

D_FEAT = 256
NUM_CORES = 2
NUM_SUBCORES = 16
LANES = 16

N_NODES_STATIC = 10000
WORDS = 8
WIN = 4096
WINDOWS = 20
E_PER_SC = WIN * WINDOWS
E_PAD = NUM_CORES * E_PER_SC
COLB = WIN // NUM_SUBCORES

_GDN = lax.GatherDimensionNumbers(
    offset_dims=(), collapsed_slice_dims=(0,), start_index_map=(0,))


def _perm(v, pat_vec):
    idx = pat_vec[:, None]
    return lax.gather(v, idx, _GDN, slice_sizes=(1,),
                      mode=lax.GatherScatterMode.PROMISE_IN_BOUNDS)


@functools.partial(
    pl.kernel,
    mesh=plsc.VectorSubcoreMesh(core_axis_name="c", subcore_axis_name="s"),
    out_type=jax.ShapeDtypeStruct((E_PAD,), jnp.float32),
    compiler_params=pltpu.CompilerParams(use_tc_tiling_on_sc=False,
                                         needs_layout_passes=False),
    scratch_types=[
        pltpu.VMEM((N_NODES_STATIC * WORDS,), jnp.int32),
        pltpu.VMEM((WIN,), jnp.int32),
        pltpu.VMEM((WIN,), jnp.int32),
        pltpu.VMEM((WIN,), jnp.int32),
        pltpu.VMEM((WIN,), jnp.int32),
        pltpu.VMEM((WIN,), jnp.float32),
        pltpu.VMEM((WIN,), jnp.float32),
        pltpu.VMEM((NUM_SUBCORES, COLB), jnp.float32),
        pltpu.VMEM((NUM_SUBCORES, COLB), jnp.float32),
        pltpu.VMEM((COLB,), jnp.float32),
        pltpu.VMEM_SHARED((2, NUM_SUBCORES, WIN), jnp.float32),
        pltpu.SemaphoreType.DMA,
        pltpu.SemaphoreType.DMA,
        pltpu.SemaphoreType.DMA,
        pltpu.SemaphoreType.DMA,
    ],
)
def _edge_dot_sc(table_hbm, dst_hbm, src_hbm, out_hbm,
                 tab_v, dw_a, sw_a, dw_b, sw_b, part_a, part_b,
                 col_a, col_b, res_v, stage,
                 sem_ia, sem_ib, sem_oa, sem_ob):
    sc = lax.axis_index("c")
    tid = lax.axis_index("s")
    ebase = sc * E_PER_SC

    pltpu.sync_copy(table_hbm.at[tid], tab_v)

    lane_iota = lax.iota(jnp.int32, LANES)
    col8 = jnp.bitwise_and(lane_iota, 7)
    lane8 = jnp.bitwise_and(lane_iota, 8)
    x4 = jnp.bitwise_xor(lane_iota, 4)
    x2 = jnp.bitwise_xor(lane_iota, 2)
    x1 = jnp.bitwise_xor(lane_iota, 1)
    m4 = jnp.bitwise_and(lane_iota, 4) == 0
    m2 = jnp.bitwise_and(lane_iota, 2) == 0
    m1 = jnp.bitwise_and(lane_iota, 1) == 0

    def merge(v0, v1, patx, mask):
        x = jnp.where(mask, v0, _perm(v1, patx))
        y = jnp.where(mask, _perm(v0, patx), v1)
        return x + y

    def issue_idx(w, dw, sw, sem):
        off = ebase + w * WIN
        pltpu.async_copy(dst_hbm.at[pl.ds(off, WIN)], dw, sem)
        pltpu.async_copy(src_hbm.at[pl.ds(off, WIN)], sw, sem)

    def wait_idx(dw, sw, sem):
        pltpu.make_async_copy(dst_hbm.at[pl.ds(0, WIN)], dw, sem).wait()
        pltpu.make_async_copy(src_hbm.at[pl.ds(0, WIN)], sw, sem).wait()

    def compute_window(dw, sw, part):
        uorder = (0, 4, 2, 6, 1, 5, 3, 7)

        def group(g, carry):
            e0 = g * LANES
            dvec = dw[pl.ds(e0, LANES)]
            svec = sw[pl.ds(e0, LANES)]
            prods = []
            for p in range(8):
                pat = lane8 + uorder[p]
                didx = _perm(dvec, pat) + col8
                sidx = _perm(svec, pat) + col8
                aw = plsc.load_gather(tab_v, [didx])
                bw = plsc.load_gather(tab_v, [sidx])
                al, ah = plsc.unpack(plsc.bitcast(aw, jnp.bfloat16),
                                     format=plsc.PackFormat.INTERLEAVED)
                bl, bh = plsc.unpack(plsc.bitcast(bw, jnp.bfloat16),
                                     format=plsc.PackFormat.INTERLEAVED)
                prods.append(al * bl + ah * bh)
            q0 = merge(prods[0], prods[1], x4, m4)
            q1 = merge(prods[2], prods[3], x4, m4)
            q2 = merge(prods[4], prods[5], x4, m4)
            q3 = merge(prods[6], prods[7], x4, m4)
            r0 = merge(q0, q1, x2, m2)
            r1 = merge(q2, q3, x2, m2)
            part[pl.ds(e0, LANES)] = merge(r0, r1, x1, m1)
            return carry

        lax.fori_loop(0, WIN // LANES, group, 0, unroll=8)

    def reduce_window(w, buf, part, col, sem):
        pltpu.make_async_copy(part, stage.at[buf, 0], sem).wait()
        plsc.subcore_barrier()
        pltpu.sync_copy(stage.at[buf, :, pl.ds(tid * COLB, COLB)], col)
        for c in range(COLB // LANES):
            acc = col[0, pl.ds(c * LANES, LANES)]
            for r in range(1, NUM_SUBCORES):
                acc = acc + col[r, pl.ds(c * LANES, LANES)]
            res_v[pl.ds(c * LANES, LANES)] = acc
        pltpu.sync_copy(res_v,
                        out_hbm.at[pl.ds(ebase + w * WIN + tid * COLB, COLB)])

    issue_idx(0, dw_a, sw_a, sem_ia)
    issue_idx(1, dw_b, sw_b, sem_ib)

    def pair_body(i, carry):
        w0 = 2 * i
        w1 = w0 + 1
        wait_idx(dw_a, sw_a, sem_ia)
        compute_window(dw_a, sw_a, part_a)
        pltpu.async_copy(part_a, stage.at[0, tid], sem_oa)

        @pl.when(i > 0)
        def _():
            reduce_window(w0 - 1, 1, part_b, col_b, sem_ob)

        issue_idx(lax.rem(w0 + 2, WINDOWS), dw_a, sw_a, sem_ia)

        wait_idx(dw_b, sw_b, sem_ib)
        compute_window(dw_b, sw_b, part_b)
        pltpu.async_copy(part_b, stage.at[1, tid], sem_ob)
        reduce_window(w0, 0, part_a, col_a, sem_oa)
        issue_idx(lax.rem(w1 + 2, WINDOWS), dw_b, sw_b, sem_ib)
        return carry

    lax.fori_loop(0, WINDOWS // 2, pair_body, 0)

    reduce_window(WINDOWS - 1, 1, part_b, col_b, sem_ob)
    wait_idx(dw_a, sw_a, sem_ia)
    wait_idx(dw_b, sw_b, sem_ib)


def kernel(node_feature, edge_dst, edge_src):
    n_nodes = node_feature.shape[0]
    n_edges = edge_dst.shape[0]
    t = node_feature.astype(jnp.bfloat16).reshape(n_nodes, 16, WORDS, 2)
    tw = lax.bitcast_convert_type(t, jnp.int32)
    tw = jnp.transpose(tw, (1, 0, 2)).reshape(16, n_nodes * WORDS)
    dst = edge_dst.astype(jnp.int32) * WORDS
    src = edge_src.astype(jnp.int32) * WORDS
    pad = E_PAD - n_edges
    dst = jnp.concatenate([dst, jnp.zeros((pad,), jnp.int32)])
    src = jnp.concatenate([src, jnp.zeros((pad,), jnp.int32)])
    out = _edge_dot_sc(tw, dst, src)
    return out[:n_edges]


# speedup vs baseline: 1.0091x; 1.0091x over previous
import functools

import jax
import jax.numpy as jnp
import numpy as np
from jax import lax
from jax.experimental import pallas as pl
from jax.experimental.pallas import tpu as pltpu
from jax.experimental.pallas import tpu_sc as plsc

D_FEAT = 256
NUM_CORES = 2
NUM_SUBCORES = 16
LANES = 16

N_NODES_STATIC = 10000
WORDS = 8
WIN = 4096
WINDOWS = 20
E_PER_SC = WIN * WINDOWS
E_PAD = NUM_CORES * E_PER_SC
COLB = WIN // NUM_SUBCORES

_GDN = lax.GatherDimensionNumbers(
    offset_dims=(), collapsed_slice_dims=(0,), start_index_map=(0,))


def _perm(v, pat_vec):
    idx = pat_vec[:, None]
    return lax.gather(v, idx, _GDN, slice_sizes=(1,),
                      mode=lax.GatherScatterMode.PROMISE_IN_BOUNDS)


@functools.partial(
    pl.kernel,
    mesh=plsc.VectorSubcoreMesh(core_axis_name="c", subcore_axis_name="s"),
    out_type=jax.ShapeDtypeStruct((E_PAD,), jnp.float32),
    compiler_params=pltpu.CompilerParams(use_tc_tiling_on_sc=False,
                                         needs_layout_passes=False),
    scratch_types=[
        pltpu.VMEM((N_NODES_STATIC * WORDS,), jnp.int32),
        pltpu.VMEM((WIN,), jnp.int32),
        pltpu.VMEM((WIN,), jnp.int32),
        pltpu.VMEM((WIN,), jnp.int32),
        pltpu.VMEM((WIN,), jnp.int32),
        pltpu.VMEM((WIN,), jnp.float32),
        pltpu.VMEM((WIN,), jnp.float32),
        pltpu.VMEM((NUM_SUBCORES, COLB), jnp.float32),
        pltpu.VMEM((NUM_SUBCORES, COLB), jnp.float32),
        pltpu.VMEM((COLB,), jnp.float32),
        pltpu.VMEM_SHARED((2, NUM_SUBCORES, WIN), jnp.float32),
        pltpu.SemaphoreType.DMA,
        pltpu.SemaphoreType.DMA,
        pltpu.SemaphoreType.DMA,
        pltpu.SemaphoreType.DMA,
    ],
)
def _edge_dot_sc(table_hbm, dst_hbm, src_hbm, out_hbm,
                 tab_v, dw_a, sw_a, dw_b, sw_b, part_a, part_b,
                 col_a, col_b, res_v, stage,
                 sem_ia, sem_ib, sem_oa, sem_ob):
    sc = lax.axis_index("c")
    tid = lax.axis_index("s")
    ebase = sc * E_PER_SC

    pltpu.sync_copy(table_hbm.at[tid], tab_v)

    lane_iota = lax.iota(jnp.int32, LANES)
    col8 = jnp.bitwise_and(lane_iota, 7)
    lane8 = jnp.bitwise_and(lane_iota, 8)
    x4 = jnp.bitwise_xor(lane_iota, 4)
    x2 = jnp.bitwise_xor(lane_iota, 2)
    x1 = jnp.bitwise_xor(lane_iota, 1)
    m4 = jnp.bitwise_and(lane_iota, 4) == 0
    m2 = jnp.bitwise_and(lane_iota, 2) == 0
    m1 = jnp.bitwise_and(lane_iota, 1) == 0

    def merge(v0, v1, patx, mask):
        x = jnp.where(mask, v0, _perm(v1, patx))
        y = jnp.where(mask, _perm(v0, patx), v1)
        return x + y

    def issue_idx(w, dw, sw, sem):
        off = ebase + w * WIN
        pltpu.async_copy(dst_hbm.at[pl.ds(off, WIN)], dw, sem)
        pltpu.async_copy(src_hbm.at[pl.ds(off, WIN)], sw, sem)

    def wait_idx(dw, sw, sem):
        pltpu.make_async_copy(dst_hbm.at[pl.ds(0, WIN)], dw, sem).wait()
        pltpu.make_async_copy(src_hbm.at[pl.ds(0, WIN)], sw, sem).wait()

    def compute_window(dw, sw, part):
        uorder = (0, 4, 2, 6, 1, 5, 3, 7)

        def group(g, carry):
            e0 = g * LANES
            dvec = dw[pl.ds(e0, LANES)]
            svec = sw[pl.ds(e0, LANES)]
            prods = []
            for p in range(8):
                pat = lane8 + uorder[p]
                didx = _perm(dvec, pat) + col8
                sidx = _perm(svec, pat) + col8
                aw = plsc.load_gather(tab_v, [didx])
                bw = plsc.load_gather(tab_v, [sidx])
                al, ah = plsc.unpack(plsc.bitcast(aw, jnp.bfloat16),
                                     format=plsc.PackFormat.INTERLEAVED)
                bl, bh = plsc.unpack(plsc.bitcast(bw, jnp.bfloat16),
                                     format=plsc.PackFormat.INTERLEAVED)
                prods.append(al * bl + ah * bh)
            q0 = merge(prods[0], prods[1], x4, m4)
            q1 = merge(prods[2], prods[3], x4, m4)
            q2 = merge(prods[4], prods[5], x4, m4)
            q3 = merge(prods[6], prods[7], x4, m4)
            r0 = merge(q0, q1, x2, m2)
            r1 = merge(q2, q3, x2, m2)
            part[pl.ds(e0, LANES)] = merge(r0, r1, x1, m1)
            return carry

        lax.fori_loop(0, WIN // LANES, group, 0, unroll=4)

    def reduce_window(w, buf, part, col, sem):
        pltpu.make_async_copy(part, stage.at[buf, 0], sem).wait()
        plsc.subcore_barrier()
        pltpu.sync_copy(stage.at[buf, :, pl.ds(tid * COLB, COLB)], col)
        for c in range(COLB // LANES):
            acc = col[0, pl.ds(c * LANES, LANES)]
            for r in range(1, NUM_SUBCORES):
                acc = acc + col[r, pl.ds(c * LANES, LANES)]
            res_v[pl.ds(c * LANES, LANES)] = acc
        pltpu.sync_copy(res_v,
                        out_hbm.at[pl.ds(ebase + w * WIN + tid * COLB, COLB)])

    issue_idx(0, dw_a, sw_a, sem_ia)
    issue_idx(1, dw_b, sw_b, sem_ib)

    def pair_body(i, carry):
        w0 = 2 * i
        w1 = w0 + 1
        wait_idx(dw_a, sw_a, sem_ia)
        compute_window(dw_a, sw_a, part_a)
        pltpu.async_copy(part_a, stage.at[0, tid], sem_oa)

        @pl.when(i > 0)
        def _():
            reduce_window(w0 - 1, 1, part_b, col_b, sem_ob)

        issue_idx(lax.rem(w0 + 2, WINDOWS), dw_a, sw_a, sem_ia)

        wait_idx(dw_b, sw_b, sem_ib)
        compute_window(dw_b, sw_b, part_b)
        pltpu.async_copy(part_b, stage.at[1, tid], sem_ob)
        reduce_window(w0, 0, part_a, col_a, sem_oa)
        issue_idx(lax.rem(w1 + 2, WINDOWS), dw_b, sw_b, sem_ib)
        return carry

    lax.fori_loop(0, WINDOWS // 2, pair_body, 0)

    reduce_window(WINDOWS - 1, 1, part_b, col_b, sem_ob)
    wait_idx(dw_a, sw_a, sem_ia)
    wait_idx(dw_b, sw_b, sem_ib)


def kernel(node_feature, edge_dst, edge_src):
    n_nodes = node_feature.shape[0]
    n_edges = edge_dst.shape[0]
    t = node_feature.astype(jnp.bfloat16).reshape(n_nodes, 16, WORDS, 2)
    tw = lax.bitcast_convert_type(t, jnp.int32)
    tw = jnp.transpose(tw, (1, 0, 2)).reshape(16, n_nodes * WORDS)
    dst = edge_dst.astype(jnp.int32) * WORDS
    src = edge_src.astype(jnp.int32) * WORDS
    pad = E_PAD - n_edges
    dst = jnp.concatenate([dst, jnp.zeros((pad,), jnp.int32)])
    src = jnp.concatenate([src, jnp.zeros((pad,), jnp.int32)])
    out = _edge_dot_sc(tw, dst, src)
    return out[:n_edges]
